# baseline (speedup 1.0000x reference)
import jax
import jax.numpy as jnp
from jax import lax
from jax.experimental import pallas as pl
from jax.experimental.pallas import tpu as pltpu
from jax.experimental.pallas import tpu_sc as plsc

B, S = 4096, 200
D = 32
NC, NS = 2, 16
NW = NC * NS
BW = B // NW
LG = BW // 16


def _emb_body(ids_hbm, wre_hbm, wim_hbm, zre_hbm, zim_hbm,
              idx_v, gb_re, gb_im, tb_re, tb_im,
              gs0, gs1, ws0, ws1):
    gsems = (gs0, gs1)
    wsems = (ws0, ws1)
    wid = lax.axis_index("s") * NC + lax.axis_index("c")
    bw0 = wid * BW

    pltpu.sync_copy(ids_hbm.at[:, pl.ds(bw0, BW)], idx_v)

    iota = lax.iota(jnp.int32, 16)

    def g_copies(s, j, mk):
        return [mk(wre_hbm.at[idx_v.at[s]], gb_re.at[j], gsems[j]),
                mk(wim_hbm.at[idx_v.at[s]], gb_im.at[j], gsems[j])]

    def w_copies(s, j, mk):
        return [mk(tb_re.at[j, :, :, pl.ds(0, BW)],
                   zre_hbm.at[s, :, wid, :, :], wsems[j]),
                mk(tb_im.at[j, :, :, pl.ds(0, BW)],
                   zim_hbm.at[s, :, wid, :, :], wsems[j])]

    def transpose(j):
        slotv = jnp.full((16,), j, jnp.int32)
        d_sub = iota & 7
        tr_lo = iota >> 3
        tr_hi = tr_lo + 2

        @plsc.parallel_loop(0, BW, unroll=8)
        def brow(b):
            bv = jnp.full((16,), 0, jnp.int32) + b
            v0 = gb_re[j, b, pl.ds(0, 16)]
            v1 = gb_re[j, b, pl.ds(16, 16)]
            plsc.store_scatter(tb_re, [slotv, tr_lo, d_sub, bv], v0)
            plsc.store_scatter(tb_re, [slotv, tr_hi, d_sub, bv], v1)
            w0 = gb_im[j, b, pl.ds(0, 16)]
            w1 = gb_im[j, b, pl.ds(16, 16)]
            plsc.store_scatter(tb_im, [slotv, tr_lo, d_sub, bv], w0)
            plsc.store_scatter(tb_im, [slotv, tr_hi, d_sub, bv], w1)

    def step(s, j, *, first, last):
        for d in g_copies(s, j, pltpu.make_async_copy):
            d.wait()
        if not last:
            g_copies(s + 1, 1 - j, pltpu.async_copy)
        if not first:
            for d in w_copies(s - 2, j, pltpu.make_async_copy):
                d.wait()
        transpose(j)
        w_copies(s, j, pltpu.async_copy)

    g_copies(0, 0, pltpu.async_copy)
    step(0, 0, first=True, last=False)
    step(1, 1, first=True, last=False)

    def steady(p, carry):
        s0 = 2 * p
        step(s0, 0, first=False, last=False)
        step(s0 + 1, 1, first=False, last=False)
        return carry

    lax.fori_loop(1, (S - 2) // 2, steady, 0)

    step(S - 2, 0, first=False, last=False)
    step(S - 1, 1, first=False, last=True)
    for d in w_copies(S - 2, 0, pltpu.make_async_copy):
        d.wait()
    for d in w_copies(S - 1, 1, pltpu.make_async_copy):
        d.wait()


V = 1000000
TC_FULL = 999936 // 128
PC = TC_FULL // NW


def _tr_body(wret_hbm, wimt_hbm, tret_hbm, timt_hbm, rre_hbm, rim_hbm,
             vb_re, vb_im, tbuf_re, tbuf_im, rs0, rs1, ts0, ts1):
    rsems = (rs0, rs1)
    wsems = (ts0, ts1)
    wid = lax.axis_index("s") * NC + lax.axis_index("c")
    base_c = PC * wid + jnp.minimum(wid, 4)

    iota = lax.iota(jnp.int32, 16)
    d_h0 = iota
    d_h1 = iota + 16

    def r_copies(i, j, mk):
        c = base_c + i
        return [mk(wret_hbm.at[:, pl.ds(128 * c, 128)], vb_re.at[j], rsems[j]),
                mk(wimt_hbm.at[:, pl.ds(128 * c, 128)], vb_im.at[j], rsems[j])]

    def w_copies(i, j, mk):
        c = base_c + i
        return [mk(tbuf_re.at[j], rre_hbm.at[pl.ds(32 * c, 32)], wsems[j]),
                mk(tbuf_im.at[j], rim_hbm.at[pl.ds(32 * c, 32)], wsems[j])]

    def transpose(j):
        slotv = jnp.full((16,), j, jnp.int32)

        @plsc.parallel_loop(0, 128, unroll=8)
        def diag(q0):
            qw = (iota + q0) & 127
            row = qw >> 2
            col0 = ((qw & 3) << 5) + iota
            col1 = col0 + 16
            v0 = plsc.load_gather(vb_re, [slotv, d_h0, qw])
            plsc.store_scatter(tbuf_re, [slotv, row, col0], v0)
            v1 = plsc.load_gather(vb_re, [slotv, d_h1, qw])
            plsc.store_scatter(tbuf_re, [slotv, row, col1], v1)
            w0 = plsc.load_gather(vb_im, [slotv, d_h0, qw])
            plsc.store_scatter(tbuf_im, [slotv, row, col0], w0)
            w1 = plsc.load_gather(vb_im, [slotv, d_h1, qw])
            plsc.store_scatter(tbuf_im, [slotv, row, col1], w1)

    def step(i, j, *, first, last):
        for d in r_copies(i, j, pltpu.make_async_copy):
            d.wait()
        if not last:
            r_copies(i + 1, 1 - j, pltpu.async_copy)
        if not first:
            for d in w_copies(i - 2, j, pltpu.make_async_copy):
                d.wait()
        transpose(j)
        w_copies(i, j, pltpu.async_copy)

    r_copies(0, 0, pltpu.async_copy)
    step(0, 0, first=True, last=False)
    step(1, 1, first=True, last=False)

    def steady(p, carry):
        i0 = 2 * p
        step(i0, 0, first=False, last=False)
        step(i0 + 1, 1, first=False, last=False)
        return carry

    lax.fori_loop(1, PC // 2 - 1, steady, 0)

    step(PC - 2, 0, first=False, last=False)
    step(PC - 1, 1, first=False, last=True)
    for d in w_copies(PC - 2, 0, pltpu.make_async_copy):
        d.wait()
    for d in w_copies(PC - 1, 1, pltpu.make_async_copy):
        d.wait()

    @pl.when(wid < 4)
    def _extra():
        c = base_c + PC
        pltpu.sync_copy(wret_hbm.at[:, pl.ds(128 * c, 128)], vb_re.at[0])
        pltpu.sync_copy(wimt_hbm.at[:, pl.ds(128 * c, 128)], vb_im.at[0])
        transpose(0)
        pltpu.sync_copy(tbuf_re.at[0], rre_hbm.at[pl.ds(32 * c, 32)])
        pltpu.sync_copy(tbuf_im.at[0], rim_hbm.at[pl.ds(32 * c, 32)])

    @pl.when(wid == 5)
    def _tail():
        pltpu.sync_copy(tret_hbm.at[:, :], vb_re.at[0])
        pltpu.sync_copy(timt_hbm.at[:, :], vb_im.at[0])
        transpose(0)
        pltpu.sync_copy(tbuf_re.at[0, pl.ds(0, 16)],
                        rre_hbm.at[pl.ds(249984, 16)])
        pltpu.sync_copy(tbuf_im.at[0, pl.ds(0, 16)],
                        rim_hbm.at[pl.ds(249984, 16)])


@jax.jit
def kernel(input_ids, W_re, W_im):
    ids_t = input_ids.T
    mesh = plsc.VectorSubcoreMesh(core_axis_name="c", subcore_axis_name="s")
    r_re, r_im = pl.kernel(
        _tr_body,
        out_type=[
            jax.ShapeDtypeStruct((V // 4, 128), jnp.float32),
            jax.ShapeDtypeStruct((V // 4, 128), jnp.float32),
        ],
        mesh=mesh,
        scratch_types=[
            pltpu.VMEM((2, D, 128), jnp.float32),
            pltpu.VMEM((2, D, 128), jnp.float32),
            pltpu.VMEM((2, D, 128), jnp.float32),
            pltpu.VMEM((2, D, 128), jnp.float32),
        ] + [pltpu.SemaphoreType.DMA] * 4,
        compiler_params=pltpu.CompilerParams(
            use_tc_tiling_on_sc=True, needs_layout_passes=False),
    )(W_re.T, W_im.T,
      jnp.pad(W_re[999936:], ((0, 64), (0, 0))).T,
      jnp.pad(W_im[999936:], ((0, 64), (0, 0))).T)
    W_re_lin = r_re.reshape(V, D)
    W_im_lin = r_im.reshape(V, D)
    z_re, z_im = pl.kernel(
        _emb_body,
        out_type=[
            jax.ShapeDtypeStruct((S, D // 8, NW, 8, BW), jnp.float32),
            jax.ShapeDtypeStruct((S, D // 8, NW, 8, BW), jnp.float32),
        ],
        mesh=mesh,
        scratch_types=[
            pltpu.VMEM((S, BW), jnp.int32),
            pltpu.VMEM((2, BW, D), jnp.float32),
            pltpu.VMEM((2, BW, D), jnp.float32),
            pltpu.VMEM((2, D // 8, 8, BW + 1), jnp.float32),
            pltpu.VMEM((2, D // 8, 8, BW + 1), jnp.float32),
        ] + [pltpu.SemaphoreType.DMA] * 4,
        compiler_params=pltpu.CompilerParams(
            use_tc_tiling_on_sc=False, needs_layout_passes=False),
    )(ids_t, W_re_lin, W_im_lin)

    def _unpack(z):
        return z.transpose(2, 4, 0, 1, 3).reshape(B, S, D)

    return (_unpack(z_re), _unpack(z_im))


# speedup vs baseline: 1.0020x; 1.0020x over previous
import jax
import jax.numpy as jnp
from jax import lax
from jax.experimental import pallas as pl
from jax.experimental.pallas import tpu as pltpu
from jax.experimental.pallas import tpu_sc as plsc

B, S = 4096, 200
D = 32
NC, NS = 2, 16
NW = NC * NS
BW = B // NW


def _emb_body(ids_hbm, wre_hbm, wim_hbm, zre_hbm, zim_hbm,
              idx_v, gb_re, gb_im, tb_re, tb_im,
              gs0, gs1, ws0, ws1):
    gsems = (gs0, gs1)
    wsems = (ws0, ws1)
    wid = lax.axis_index("s") * NC + lax.axis_index("c")
    bw0 = wid * BW

    pltpu.sync_copy(ids_hbm.at[:, pl.ds(bw0, BW)], idx_v)

    iota = lax.iota(jnp.int32, 16)

    def g_copies(s, j, mk):
        return [mk(wre_hbm.at[idx_v.at[s]], gb_re.at[j], gsems[j]),
                mk(wim_hbm.at[idx_v.at[s]], gb_im.at[j], gsems[j])]

    def w_copies(s, j, mk):
        return [mk(tb_re.at[j, :, :, pl.ds(0, BW)],
                   zre_hbm.at[s, :, wid, :, :], wsems[j]),
                mk(tb_im.at[j, :, :, pl.ds(0, BW)],
                   zim_hbm.at[s, :, wid, :, :], wsems[j])]

    def transpose(j):
        slotv = jnp.full((16,), j, jnp.int32)
        d_sub = iota & 7
        tr_lo = iota >> 3
        tr_hi = tr_lo + 2

        @plsc.parallel_loop(0, BW, unroll=8)
        def brow(b):
            bv = jnp.full((16,), 0, jnp.int32) + b
            v0 = gb_re[j, b, pl.ds(0, 16)]
            v1 = gb_re[j, b, pl.ds(16, 16)]
            plsc.store_scatter(tb_re, [slotv, tr_lo, d_sub, bv], v0)
            plsc.store_scatter(tb_re, [slotv, tr_hi, d_sub, bv], v1)
            w0 = gb_im[j, b, pl.ds(0, 16)]
            w1 = gb_im[j, b, pl.ds(16, 16)]
            plsc.store_scatter(tb_im, [slotv, tr_lo, d_sub, bv], w0)
            plsc.store_scatter(tb_im, [slotv, tr_hi, d_sub, bv], w1)

    def step(s, j, *, first, last):
        for d in g_copies(s, j, pltpu.make_async_copy):
            d.wait()
        if not last:
            g_copies(s + 1, 1 - j, pltpu.async_copy)
        if not first:
            for d in w_copies(s - 2, j, pltpu.make_async_copy):
                d.wait()
        transpose(j)
        w_copies(s, j, pltpu.async_copy)

    g_copies(0, 0, pltpu.async_copy)
    step(0, 0, first=True, last=False)
    step(1, 1, first=True, last=False)

    def steady(p, carry):
        s0 = 2 * p
        step(s0, 0, first=False, last=False)
        step(s0 + 1, 1, first=False, last=False)
        return carry

    lax.fori_loop(1, (S - 2) // 2, steady, 0)

    step(S - 2, 0, first=False, last=False)
    step(S - 1, 1, first=False, last=True)
    for d in w_copies(S - 2, 0, pltpu.make_async_copy):
        d.wait()
    for d in w_copies(S - 1, 1, pltpu.make_async_copy):
        d.wait()


V = 1000000
TC_FULL = 999936 // 128
PC = TC_FULL // NW


def _tr_body(wret_hbm, wimt_hbm, tret_hbm, timt_hbm, rre_hbm, rim_hbm,
             vb_re, vb_im, tbuf_re, tbuf_im, rs0, rs1, ts0, ts1):
    rsems = (rs0, rs1)
    wsems = (ts0, ts1)
    wid = lax.axis_index("s") * NC + lax.axis_index("c")
    base_c = PC * wid + jnp.minimum(wid, 4)

    iota = lax.iota(jnp.int32, 16)
    d_h0 = iota
    d_h1 = iota + 16

    def r_copies(i, j, mk):
        c = base_c + i
        return [mk(wret_hbm.at[:, pl.ds(128 * c, 128)], vb_re.at[j], rsems[j]),
                mk(wimt_hbm.at[:, pl.ds(128 * c, 128)], vb_im.at[j], rsems[j])]

    def w_copies(i, j, mk):
        c = base_c + i
        return [mk(tbuf_re.at[j], rre_hbm.at[pl.ds(32 * c, 32)], wsems[j]),
                mk(tbuf_im.at[j], rim_hbm.at[pl.ds(32 * c, 32)], wsems[j])]

    def transpose(j):
        slotv = jnp.full((16,), j, jnp.int32)

        @plsc.parallel_loop(0, 128, unroll=8)
        def diag(q0):
            qw = (iota + q0) & 127
            row = qw >> 2
            col0 = ((qw & 3) << 5) + iota
            col1 = col0 + 16
            v0 = plsc.load_gather(vb_re, [slotv, d_h0, qw])
            plsc.store_scatter(tbuf_re, [slotv, row, col0], v0)
            v1 = plsc.load_gather(vb_re, [slotv, d_h1, qw])
            plsc.store_scatter(tbuf_re, [slotv, row, col1], v1)
            w0 = plsc.load_gather(vb_im, [slotv, d_h0, qw])
            plsc.store_scatter(tbuf_im, [slotv, row, col0], w0)
            w1 = plsc.load_gather(vb_im, [slotv, d_h1, qw])
            plsc.store_scatter(tbuf_im, [slotv, row, col1], w1)

    def step(i, j, *, first, last):
        for d in r_copies(i, j, pltpu.make_async_copy):
            d.wait()
        if not last:
            r_copies(i + 1, 1 - j, pltpu.async_copy)
        if not first:
            for d in w_copies(i - 2, j, pltpu.make_async_copy):
                d.wait()
        transpose(j)
        w_copies(i, j, pltpu.async_copy)

    r_copies(0, 0, pltpu.async_copy)
    step(0, 0, first=True, last=False)
    step(1, 1, first=True, last=False)

    def steady(p, carry):
        i0 = 2 * p
        step(i0, 0, first=False, last=False)
        step(i0 + 1, 1, first=False, last=False)
        return carry

    lax.fori_loop(1, PC // 2 - 1, steady, 0)

    step(PC - 2, 0, first=False, last=False)
    step(PC - 1, 1, first=False, last=True)
    for d in w_copies(PC - 2, 0, pltpu.make_async_copy):
        d.wait()
    for d in w_copies(PC - 1, 1, pltpu.make_async_copy):
        d.wait()

    @pl.when(wid < 4)
    def _extra():
        c = base_c + PC
        pltpu.sync_copy(wret_hbm.at[:, pl.ds(128 * c, 128)], vb_re.at[0])
        pltpu.sync_copy(wimt_hbm.at[:, pl.ds(128 * c, 128)], vb_im.at[0])
        transpose(0)
        pltpu.sync_copy(tbuf_re.at[0], rre_hbm.at[pl.ds(32 * c, 32)])
        pltpu.sync_copy(tbuf_im.at[0], rim_hbm.at[pl.ds(32 * c, 32)])

    @pl.when(wid == 5)
    def _tail():
        pltpu.sync_copy(tret_hbm.at[:, :], vb_re.at[0])
        pltpu.sync_copy(timt_hbm.at[:, :], vb_im.at[0])
        transpose(0)
        pltpu.sync_copy(tbuf_re.at[0, pl.ds(0, 16)],
                        rre_hbm.at[pl.ds(249984, 16)])
        pltpu.sync_copy(tbuf_im.at[0, pl.ds(0, 16)],
                        rim_hbm.at[pl.ds(249984, 16)])


@jax.jit
def kernel(input_ids, W_re, W_im):
    ids_t = input_ids.T
    mesh = plsc.VectorSubcoreMesh(core_axis_name="c", subcore_axis_name="s")
    r_re, r_im = pl.kernel(
        _tr_body,
        out_type=[
            jax.ShapeDtypeStruct((V // 4, 128), jnp.float32),
            jax.ShapeDtypeStruct((V // 4, 128), jnp.float32),
        ],
        mesh=mesh,
        scratch_types=[
            pltpu.VMEM((2, D, 128), jnp.float32),
            pltpu.VMEM((2, D, 128), jnp.float32),
            pltpu.VMEM((2, D, 128), jnp.float32),
            pltpu.VMEM((2, D, 128), jnp.float32),
        ] + [pltpu.SemaphoreType.DMA] * 4,
        compiler_params=pltpu.CompilerParams(
            use_tc_tiling_on_sc=True, needs_layout_passes=False),
    )(W_re.T, W_im.T,
      jnp.pad(W_re[999936:], ((0, 64), (0, 0))).T,
      jnp.pad(W_im[999936:], ((0, 64), (0, 0))).T)
    W_re_lin = r_re.reshape(V, D)
    W_im_lin = r_im.reshape(V, D)
    z_re, z_im = pl.kernel(
        _emb_body,
        out_type=[
            jax.ShapeDtypeStruct((S, D // 8, NW, 8, BW), jnp.float32),
            jax.ShapeDtypeStruct((S, D // 8, NW, 8, BW), jnp.float32),
        ],
        mesh=mesh,
        scratch_types=[
            pltpu.VMEM((S, BW), jnp.int32),
            pltpu.VMEM((2, BW, D), jnp.float32),
            pltpu.VMEM((2, BW, D), jnp.float32),
            pltpu.VMEM((2, D // 8, 8, BW + 1), jnp.float32),
            pltpu.VMEM((2, D // 8, 8, BW + 1), jnp.float32),
        ] + [pltpu.SemaphoreType.DMA] * 4,
        compiler_params=pltpu.CompilerParams(
            use_tc_tiling_on_sc=False, needs_layout_passes=False),
    )(ids_t, W_re_lin, W_im_lin)

    def _unpack(z):
        return z.transpose(2, 4, 0, 1, 3).reshape(B, S, D)

    return (_unpack(z_re), _unpack(z_im))
